# baseline (speedup 1.0000x reference)
import functools

import jax
import jax.numpy as jnp
from jax import lax
from jax.experimental import pallas as pl
from jax.experimental.pallas import tpu as pltpu
from jax.experimental.pallas import tpu_sc as plsc

B = 4
NP = 10000
NNZ = 160000
N = B * NP

NC = 2
NS = 16
L = 16

BLK = 128
NB = NNZ // BLK
NB_TILE = 78
KP = 4
SWP = KP * BLK
NPASS = 20
TAIL_OFF = NS * NB_TILE * BLK
TAIL_W = NNZ - TAIL_OFF
NP_PAD = 10240
SLICE = NP_PAD // NS
MSE_CHUNK = 1280

PASS_W = [SWP] * 19 + [2 * BLK]
PASS_OFF = [p * SWP for p in range(NPASS)]


def _sc_body(er_hbm, ei_hbm, y_hbm, rows_hbm, cols_hbm, vr_hbm, vi_hbm,
             br_hbm, bi_hbm,
             out_hbm, exch_hbm,
             str0, stc0, stvr0, stvi0, str1, stc1, stvr1, stvi1,
             accr0, acci0, accr1, acci1, xr2, xi2,
             tmp, m0, m1, myf, brv, biv, outv,
             sem_st0, sem_st1, sem_x, sem_b, sem_mse, sem_t):
  c = lax.axis_index("c")
  s = lax.axis_index("s")

  zeros = jnp.zeros((L,), jnp.float32)
  iota = lax.broadcasted_iota(jnp.int32, (L,), 0)
  wid = c * NS + s

  st = [(str0, stc0, stvr0, stvi0, sem_st0),
        (str1, stc1, stvr1, stvi1, sem_st1)]
  accs = [(accr0, acci0), (accr1, acci1)]

  x_copies = [
      pltpu.async_copy(er_hbm.at[pl.ds(c * 2 * NP, 2 * NP)], xr2, sem_x),
      pltpu.async_copy(ei_hbm.at[pl.ds(c * 2 * NP, 2 * NP)], xi2, sem_x),
  ]
  mse_off = pl.multiple_of(jnp.minimum(wid * MSE_CHUNK, N - MSE_CHUNK), 8)
  mse_copies = [
      pltpu.async_copy(er_hbm.at[pl.ds(mse_off, MSE_CHUNK)], m0, sem_mse),
      pltpu.async_copy(ei_hbm.at[pl.ds(mse_off, MSE_CHUNK)], m1, sem_mse),
      pltpu.async_copy(
          y_hbm.at[pl.ds(2 * mse_off, 2 * MSE_CHUNK)], myf, sem_mse),
  ]

  col0 = s * (NB_TILE * BLK)

  def fire_pass(p):
    r, co, vr_, vi_, sem = st[p % 2]
    w = PASS_W[p]
    off = pl.multiple_of(col0 + PASS_OFF[p], BLK)
    return [
        pltpu.async_copy(rows_hbm.at[:, pl.ds(off, w)],
                         r.at[:, pl.ds(0, w)], sem),
        pltpu.async_copy(cols_hbm.at[:, pl.ds(off, w)],
                         co.at[:, pl.ds(0, w)], sem),
        pltpu.async_copy(vr_hbm.at[:, pl.ds(off, w)],
                         vr_.at[:, pl.ds(0, w)], sem),
        pltpu.async_copy(vi_hbm.at[:, pl.ds(off, w)],
                         vi_.at[:, pl.ds(0, w)], sem),
    ]

  in_flight = {0: fire_pass(0)}

  @plsc.parallel_loop(0, NP // L, unroll=5)
  def _(k):
    off = pl.ds(k * L, L)
    accr0[off] = zeros
    acci0[off] = zeros
    accr1[off] = zeros
    acci1[off] = zeros

  with jax.named_scope("x_wait"):
    for cp in x_copies:
      cp.wait()

  def phase1_block(r_ref, c_ref, vr_ref, vi_ref, nvregs):
    for b_local in range(2):
      brow16 = jnp.full((L,), 2 * c + b_local, jnp.int32)
      acc_r, acc_i = accs[b_local]
      xbase = b_local * NP

      @plsc.parallel_loop(0, nvregs, unroll=4)
      def _(t):
        idx16 = t * L + iota
        rowg = plsc.load_gather(r_ref, [brow16, idx16])
        colg = plsc.load_gather(c_ref, [brow16, idx16])
        wr = plsc.load_gather(vr_ref, [brow16, idx16])
        wi = plsc.load_gather(vi_ref, [brow16, idx16])
        xcr = plsc.load_gather(xr2, [colg + xbase])
        xci = plsc.load_gather(xi2, [colg + xbase])
        ar = wr * xcr - wi * xci
        ai = wr * xci + wi * xcr
        plsc.addupdate_scatter(acc_r, [rowg], ar)
        plsc.addupdate_scatter(acc_i, [rowg], ai)

  for p in range(NPASS):
    if p + 1 < NPASS:
      in_flight[p + 1] = fire_pass(p + 1)
    with jax.named_scope("st_wait"):
      for cp in in_flight.pop(p):
        cp.wait()
    r_ref, c_ref, vr_ref, vi_ref, _ = st[p % 2]
    with jax.named_scope("phase1"):
      phase1_block(r_ref, c_ref, vr_ref, vi_ref, PASS_W[p] // L)

  @pl.when(s == 0)
  def _():
    pltpu.sync_copy(rows_hbm.at[:, pl.ds(TAIL_OFF, TAIL_W)],
                    str0.at[:, pl.ds(0, TAIL_W)])
    pltpu.sync_copy(cols_hbm.at[:, pl.ds(TAIL_OFF, TAIL_W)],
                    stc0.at[:, pl.ds(0, TAIL_W)])
    pltpu.sync_copy(vr_hbm.at[:, pl.ds(TAIL_OFF, TAIL_W)],
                    stvr0.at[:, pl.ds(0, TAIL_W)])
    pltpu.sync_copy(vi_hbm.at[:, pl.ds(TAIL_OFF, TAIL_W)],
                    stvi0.at[:, pl.ds(0, TAIL_W)])
    phase1_block(str0, stc0, stvr0, stvi0, TAIL_W // L)

  row_base = s * SLICE
  b_off = pl.multiple_of(jnp.minimum(row_base, NP - SLICE), 8)

  def b_slices(bi):
    boff = pl.multiple_of(bi * NP, 8) + b_off
    return (br_hbm.at[pl.ds(boff, SLICE)], bi_hbm.at[pl.ds(boff, SLICE)])

  def pub_off(tile, slot):
    return pl.multiple_of(((c * NS + tile) * 4 + slot) * NP, 8)

  with jax.named_scope("publish"):
    pltpu.sync_copy(accr0, exch_hbm.at[pl.ds(pub_off(s, 0), NP)])
    pltpu.sync_copy(acci0, exch_hbm.at[pl.ds(pub_off(s, 1), NP)])
    pltpu.sync_copy(accr1, exch_hbm.at[pl.ds(pub_off(s, 2), NP)])
    pltpu.sync_copy(acci1, exch_hbm.at[pl.ds(pub_off(s, 3), NP)])
    plsc.subcore_barrier()

  b_shift = row_base - b_off
  loss_vec = zeros
  for b_local in range(2):
    bi = 2 * c + b_local
    src_r, src_i = b_slices(bi)
    with jax.named_scope("b_copy"):
      bcp = [pltpu.async_copy(src_r, brv, sem_b),
             pltpu.async_copy(src_i, biv, sem_b)]
    for comp in range(2):
      slot = 2 * b_local + comp
      with jax.named_scope("tmp_copy"):
        tcp = [
            pltpu.async_copy(
                exch_hbm.at[pl.ds(
                    pl.multiple_of(pub_off(t, slot) + b_off, 8), SLICE)],
                tmp.at[pl.ds(t * SLICE, SLICE)], sem_t)
            for t in range(NS)
        ]
        for cp in tcp:
          cp.wait()
      if comp == 0:
        for cp in bcp:
          cp.wait()
      b_ref = brv if comp == 0 else biv

      @plsc.parallel_loop(0, SLICE // L, carry=loss_vec)
      def res_body(k, acc):
        koff = jnp.minimum(b_shift + k * L, SLICE - L)
        ax = tmp[pl.ds(koff, L)]
        for t in range(1, NS):
          ax = ax + tmp[pl.ds(t * SLICE + koff, L)]
        r = b_ref[pl.ds(koff, L)] - ax
        row_vec = row_base + k * L + iota
        return acc + jnp.where(row_vec < NP, r * r, jnp.float32(0.0))
      loss_vec = res_body

  mse_shift = wid * MSE_CHUNK - mse_off
  with jax.named_scope("mse_wait"):
    for cp in mse_copies:
      cp.wait()

  @plsc.parallel_loop(0, MSE_CHUNK // L, carry=loss_vec)
  def mse_body(k, acc):
    roff = jnp.minimum(mse_shift + k * L, MSE_CHUNK - L)
    off = pl.ds(roff, L)
    idx16 = roff + iota
    yr = plsc.load_gather(myf, [2 * idx16])
    yi = plsc.load_gather(myf, [2 * idx16 + 1])
    dr = m0[off] - yr
    di = m1[off] - yi
    contrib = dr * dr + di * di
    elem = wid * MSE_CHUNK + k * L + iota
    return acc + jnp.where(elem < N, contrib, jnp.float32(0.0))
  loss_vec = mse_body

  outv[...] = loss_vec * jnp.float32(0.5 / N)
  pltpu.sync_copy(outv, out_hbm.at[wid])


@jax.jit
def _run(er, ei, y, rows, cols, vr, vi, br, bi):
  mesh = plsc.VectorSubcoreMesh(
      core_axis_name="c", subcore_axis_name="s",
      num_cores=NC, num_subcores=NS)
  f = pl.kernel(
      _sc_body,
      out_type=(jax.ShapeDtypeStruct((NC * NS, L), jnp.float32),
                jax.ShapeDtypeStruct((NC * NS * 4 * NP,), jnp.float32)),
      mesh=mesh,
      compiler_params=pltpu.CompilerParams(needs_layout_passes=False),
      scratch_types=[
          pltpu.VMEM((B, SWP), jnp.int32),
          pltpu.VMEM((B, SWP), jnp.int32),
          pltpu.VMEM((B, SWP), jnp.float32),
          pltpu.VMEM((B, SWP), jnp.float32),
          pltpu.VMEM((B, SWP), jnp.int32),
          pltpu.VMEM((B, SWP), jnp.int32),
          pltpu.VMEM((B, SWP), jnp.float32),
          pltpu.VMEM((B, SWP), jnp.float32),
          pltpu.VMEM((NP,), jnp.float32),
          pltpu.VMEM((NP,), jnp.float32),
          pltpu.VMEM((NP,), jnp.float32),
          pltpu.VMEM((NP,), jnp.float32),
          pltpu.VMEM((2 * NP,), jnp.float32),
          pltpu.VMEM((2 * NP,), jnp.float32),
          pltpu.VMEM((NS * SLICE,), jnp.float32),
          pltpu.VMEM((MSE_CHUNK,), jnp.float32),
          pltpu.VMEM((MSE_CHUNK,), jnp.float32),
          pltpu.VMEM((2 * MSE_CHUNK,), jnp.float32),
          pltpu.VMEM((SLICE,), jnp.float32),
          pltpu.VMEM((SLICE,), jnp.float32),
          pltpu.VMEM((L,), jnp.float32),
          pltpu.SemaphoreType.DMA,
          pltpu.SemaphoreType.DMA,
          pltpu.SemaphoreType.DMA,
          pltpu.SemaphoreType.DMA,
          pltpu.SemaphoreType.DMA,
          pltpu.SemaphoreType.DMA,
      ],
  )
  partials, _ = f(er, ei, y, rows, cols, vr, vi, br, bi)
  return partials


def kernel(E_real, E_imag, batch_y, k_all, node_batch, A_rows, A_cols,
           A_vals_real, A_vals_imag, b_real, b_imag):
  del k_all, node_batch
  partials = _run(E_real, E_imag, batch_y.reshape(-1), A_rows, A_cols,
                  A_vals_real, A_vals_imag,
                  b_real.reshape(-1), b_imag.reshape(-1))
  return jnp.sum(partials)


# speedup vs baseline: 1.0228x; 1.0228x over previous
import functools

import jax
import jax.numpy as jnp
from jax import lax
from jax.experimental import pallas as pl
from jax.experimental.pallas import tpu as pltpu
from jax.experimental.pallas import tpu_sc as plsc

B = 4
NP = 10000
NNZ = 160000
N = B * NP

NC = 2
NS = 16
L = 16

BLK = 128
NB = NNZ // BLK
NB_TILE = 78
KP = 3
SWP = KP * BLK
NPASS = 26
TAIL_OFF = NS * NB_TILE * BLK
TAIL_W = NNZ - TAIL_OFF
NP_PAD = 10240
SLICE = NP_PAD // NS
MSE_CHUNK = 1280


def _sc_body(er_hbm, ei_hbm, y_hbm, rows_hbm, cols_hbm, vr_hbm, vi_hbm,
             br_hbm, bi_hbm,
             out_hbm, exch_hbm,
             str0, stc0, stvr0, stvi0, str1, stc1, stvr1, stvi1,
             accr0, acci0, accr1, acci1, xr2, xi2,
             tmp, m0, m1, myf, brv, biv, outv,
             sem_st0, sem_st1, sem_x, sem_b, sem_mse, sem_t):
  c = lax.axis_index("c")
  s = lax.axis_index("s")

  zeros = jnp.zeros((L,), jnp.float32)
  iota = lax.broadcasted_iota(jnp.int32, (L,), 0)
  wid = c * NS + s

  st = [(str0, stc0, stvr0, stvi0, sem_st0),
        (str1, stc1, stvr1, stvi1, sem_st1)]
  accs = [(accr0, acci0), (accr1, acci1)]

  x_copies = [
      pltpu.async_copy(er_hbm.at[pl.ds(c * 2 * NP, 2 * NP)], xr2, sem_x),
      pltpu.async_copy(ei_hbm.at[pl.ds(c * 2 * NP, 2 * NP)], xi2, sem_x),
  ]
  mse_off = pl.multiple_of(jnp.minimum(wid * MSE_CHUNK, N - MSE_CHUNK), 8)
  mse_copies = [
      pltpu.async_copy(er_hbm.at[pl.ds(mse_off, MSE_CHUNK)], m0, sem_mse),
      pltpu.async_copy(ei_hbm.at[pl.ds(mse_off, MSE_CHUNK)], m1, sem_mse),
      pltpu.async_copy(
          y_hbm.at[pl.ds(2 * mse_off, 2 * MSE_CHUNK)], myf, sem_mse),
  ]

  col0 = s * (NB_TILE * BLK)

  def pass_copies(p, parity):
    r, co, vr_, vi_, sem = st[parity]
    off = pl.multiple_of(col0 + p * SWP, BLK)
    return [
        pltpu.make_async_copy(rows_hbm.at[:, pl.ds(off, SWP)], r, sem),
        pltpu.make_async_copy(cols_hbm.at[:, pl.ds(off, SWP)], co, sem),
        pltpu.make_async_copy(vr_hbm.at[:, pl.ds(off, SWP)], vr_, sem),
        pltpu.make_async_copy(vi_hbm.at[:, pl.ds(off, SWP)], vi_, sem),
    ]

  def fire_pass(p, parity):
    for cp in pass_copies(p, parity):
      cp.start()

  def wait_pass(p, parity):
    for cp in pass_copies(p, parity):
      cp.wait()

  fire_pass(0, 0)

  @plsc.parallel_loop(0, NP // L, unroll=5)
  def _(k):
    off = pl.ds(k * L, L)
    accr0[off] = zeros
    acci0[off] = zeros
    accr1[off] = zeros
    acci1[off] = zeros

  with jax.named_scope("x_wait"):
    for cp in x_copies:
      cp.wait()

  def phase1_block(r_ref, c_ref, vr_ref, vi_ref, nvregs):
    for b_local in range(2):
      brow16 = jnp.full((L,), 2 * c + b_local, jnp.int32)
      acc_r, acc_i = accs[b_local]
      xbase = b_local * NP

      @plsc.parallel_loop(0, nvregs, unroll=4)
      def _(t):
        idx16 = t * L + iota
        rowg = plsc.load_gather(r_ref, [brow16, idx16])
        colg = plsc.load_gather(c_ref, [brow16, idx16])
        wr = plsc.load_gather(vr_ref, [brow16, idx16])
        wi = plsc.load_gather(vi_ref, [brow16, idx16])
        xcr = plsc.load_gather(xr2, [colg + xbase])
        xci = plsc.load_gather(xi2, [colg + xbase])
        ar = wr * xcr - wi * xci
        ai = wr * xci + wi * xcr
        plsc.addupdate_scatter(acc_r, [rowg], ar)
        plsc.addupdate_scatter(acc_i, [rowg], ai)

  def pass_pair(k, _):
    p0 = k * 2
    fire_pass(p0 + 1, 1)
    with jax.named_scope("st_wait"):
      wait_pass(p0, 0)
    with jax.named_scope("phase1"):
      phase1_block(str0, stc0, stvr0, stvi0, SWP // L)

    @pl.when(p0 + 2 < NPASS)
    def _():
      fire_pass(p0 + 2, 0)

    with jax.named_scope("st_wait"):
      wait_pass(p0 + 1, 1)
    with jax.named_scope("phase1"):
      phase1_block(str1, stc1, stvr1, stvi1, SWP // L)
    return 0

  lax.fori_loop(0, NPASS // 2, pass_pair, 0)

  @pl.when(s == 0)
  def _():
    pltpu.sync_copy(rows_hbm.at[:, pl.ds(TAIL_OFF, TAIL_W)],
                    str0.at[:, pl.ds(0, TAIL_W)])
    pltpu.sync_copy(cols_hbm.at[:, pl.ds(TAIL_OFF, TAIL_W)],
                    stc0.at[:, pl.ds(0, TAIL_W)])
    pltpu.sync_copy(vr_hbm.at[:, pl.ds(TAIL_OFF, TAIL_W)],
                    stvr0.at[:, pl.ds(0, TAIL_W)])
    pltpu.sync_copy(vi_hbm.at[:, pl.ds(TAIL_OFF, TAIL_W)],
                    stvi0.at[:, pl.ds(0, TAIL_W)])
    phase1_block(str0, stc0, stvr0, stvi0, TAIL_W // L)

  row_base = s * SLICE
  b_off = pl.multiple_of(jnp.minimum(row_base, NP - SLICE), 8)

  def b_slices(bi):
    boff = pl.multiple_of(bi * NP, 8) + b_off
    return (br_hbm.at[pl.ds(boff, SLICE)], bi_hbm.at[pl.ds(boff, SLICE)])

  def pub_off(tile, slot):
    return pl.multiple_of(((c * NS + tile) * 4 + slot) * NP, 8)

  with jax.named_scope("publish"):
    pubs = [
        pltpu.async_copy(accr0, exch_hbm.at[pl.ds(pub_off(s, 0), NP)], sem_b),
        pltpu.async_copy(acci0, exch_hbm.at[pl.ds(pub_off(s, 1), NP)], sem_b),
        pltpu.async_copy(accr1, exch_hbm.at[pl.ds(pub_off(s, 2), NP)], sem_b),
        pltpu.async_copy(acci1, exch_hbm.at[pl.ds(pub_off(s, 3), NP)], sem_b),
    ]
    for cp in pubs:
      cp.wait()
    plsc.subcore_barrier()

  b_shift = row_base - b_off
  loss_vec = zeros
  for b_local in range(2):
    bi = 2 * c + b_local
    src_r, src_i = b_slices(bi)
    with jax.named_scope("b_copy"):
      bcp = [pltpu.async_copy(src_r, brv, sem_b),
             pltpu.async_copy(src_i, biv, sem_b)]
    for comp in range(2):
      slot = 2 * b_local + comp
      with jax.named_scope("tmp_copy"):
        tcp = [
            pltpu.async_copy(
                exch_hbm.at[pl.ds(
                    pl.multiple_of(pub_off(t, slot) + b_off, 8), SLICE)],
                tmp.at[pl.ds(t * SLICE, SLICE)], sem_t)
            for t in range(NS)
        ]
        for cp in tcp:
          cp.wait()
      if comp == 0:
        for cp in bcp:
          cp.wait()
      b_ref = brv if comp == 0 else biv

      @plsc.parallel_loop(0, SLICE // L, carry=loss_vec)
      def res_body(k, acc):
        koff = jnp.minimum(b_shift + k * L, SLICE - L)
        ax = tmp[pl.ds(koff, L)]
        for t in range(1, NS):
          ax = ax + tmp[pl.ds(t * SLICE + koff, L)]
        r = b_ref[pl.ds(koff, L)] - ax
        row_vec = row_base + k * L + iota
        return acc + jnp.where(row_vec < NP, r * r, jnp.float32(0.0))
      loss_vec = res_body

  mse_shift = wid * MSE_CHUNK - mse_off
  with jax.named_scope("mse_wait"):
    for cp in mse_copies:
      cp.wait()

  @plsc.parallel_loop(0, MSE_CHUNK // L, carry=loss_vec)
  def mse_body(k, acc):
    roff = jnp.minimum(mse_shift + k * L, MSE_CHUNK - L)
    off = pl.ds(roff, L)
    idx16 = roff + iota
    yr = plsc.load_gather(myf, [2 * idx16])
    yi = plsc.load_gather(myf, [2 * idx16 + 1])
    dr = m0[off] - yr
    di = m1[off] - yi
    contrib = dr * dr + di * di
    elem = wid * MSE_CHUNK + k * L + iota
    return acc + jnp.where(elem < N, contrib, jnp.float32(0.0))
  loss_vec = mse_body

  outv[...] = loss_vec * jnp.float32(0.5 / N)
  pltpu.sync_copy(outv, out_hbm.at[wid])


@jax.jit
def _run(er, ei, y, rows, cols, vr, vi, br, bi):
  mesh = plsc.VectorSubcoreMesh(
      core_axis_name="c", subcore_axis_name="s",
      num_cores=NC, num_subcores=NS)
  f = pl.kernel(
      _sc_body,
      out_type=(jax.ShapeDtypeStruct((NC * NS, L), jnp.float32),
                jax.ShapeDtypeStruct((NC * NS * 4 * NP,), jnp.float32)),
      mesh=mesh,
      compiler_params=pltpu.CompilerParams(needs_layout_passes=False),
      scratch_types=[
          pltpu.VMEM((B, SWP), jnp.int32),
          pltpu.VMEM((B, SWP), jnp.int32),
          pltpu.VMEM((B, SWP), jnp.float32),
          pltpu.VMEM((B, SWP), jnp.float32),
          pltpu.VMEM((B, SWP), jnp.int32),
          pltpu.VMEM((B, SWP), jnp.int32),
          pltpu.VMEM((B, SWP), jnp.float32),
          pltpu.VMEM((B, SWP), jnp.float32),
          pltpu.VMEM((NP,), jnp.float32),
          pltpu.VMEM((NP,), jnp.float32),
          pltpu.VMEM((NP,), jnp.float32),
          pltpu.VMEM((NP,), jnp.float32),
          pltpu.VMEM((2 * NP,), jnp.float32),
          pltpu.VMEM((2 * NP,), jnp.float32),
          pltpu.VMEM((NS * SLICE,), jnp.float32),
          pltpu.VMEM((MSE_CHUNK,), jnp.float32),
          pltpu.VMEM((MSE_CHUNK,), jnp.float32),
          pltpu.VMEM((2 * MSE_CHUNK,), jnp.float32),
          pltpu.VMEM((SLICE,), jnp.float32),
          pltpu.VMEM((SLICE,), jnp.float32),
          pltpu.VMEM((L,), jnp.float32),
          pltpu.SemaphoreType.DMA,
          pltpu.SemaphoreType.DMA,
          pltpu.SemaphoreType.DMA,
          pltpu.SemaphoreType.DMA,
          pltpu.SemaphoreType.DMA,
          pltpu.SemaphoreType.DMA,
      ],
  )
  partials, _ = f(er, ei, y, rows, cols, vr, vi, br, bi)
  return partials


def kernel(E_real, E_imag, batch_y, k_all, node_batch, A_rows, A_cols,
           A_vals_real, A_vals_imag, b_real, b_imag):
  del k_all, node_batch
  partials = _run(E_real, E_imag, batch_y.reshape(-1), A_rows, A_cols,
                  A_vals_real, A_vals_imag,
                  b_real.reshape(-1), b_imag.reshape(-1))
  return jnp.sum(partials)


# speedup vs baseline: 1.0237x; 1.0008x over previous
import functools

import jax
import jax.numpy as jnp
from jax import lax
from jax.experimental import pallas as pl
from jax.experimental.pallas import tpu as pltpu
from jax.experimental.pallas import tpu_sc as plsc

B = 4
NP = 10000
NNZ = 160000
N = B * NP

NC = 2
NS = 16
L = 16

BLK = 128
NB = NNZ // BLK
NB_TILE = 78
KP = 3
SWP = KP * BLK
NPASS = 26
TAIL_OFF = NS * NB_TILE * BLK
TAIL_W = NNZ - TAIL_OFF
NP_PAD = 10240
SLICE = NP_PAD // NS
MSE_CHUNK = 1280


def _sc_body(er_hbm, ei_hbm, y_hbm, rows_hbm, cols_hbm, vr_hbm, vi_hbm,
             br_hbm, bi_hbm,
             out_hbm,
             exch_hbm,
             str0, stc0, stvr0, stvi0, str1, stc1, stvr1, stvi1,
             accr0, acci0, accr1, acci1, xr2, xi2,
             tmp, m0, m1, myf, brv, biv, outv,
             sem_st0, sem_st1, sem_x, sem_b, sem_mse, sem_t):
  c = lax.axis_index("c")
  s = lax.axis_index("s")

  zeros = jnp.zeros((L,), jnp.float32)
  iota = lax.broadcasted_iota(jnp.int32, (L,), 0)
  wid = c * NS + s

  st = [(str0, stc0, stvr0, stvi0, sem_st0),
        (str1, stc1, stvr1, stvi1, sem_st1)]
  accs = [(accr0, acci0), (accr1, acci1)]

  x_copies = [
      pltpu.async_copy(er_hbm.at[pl.ds(c * 2 * NP, 2 * NP)], xr2, sem_x),
      pltpu.async_copy(ei_hbm.at[pl.ds(c * 2 * NP, 2 * NP)], xi2, sem_x),
  ]
  mse_off = pl.multiple_of(jnp.minimum(wid * MSE_CHUNK, N - MSE_CHUNK), 8)
  mse_copies = [
      pltpu.async_copy(er_hbm.at[pl.ds(mse_off, MSE_CHUNK)], m0, sem_mse),
      pltpu.async_copy(ei_hbm.at[pl.ds(mse_off, MSE_CHUNK)], m1, sem_mse),
      pltpu.async_copy(
          y_hbm.at[pl.ds(2 * mse_off, 2 * MSE_CHUNK)], myf, sem_mse),
  ]

  col0 = s * (NB_TILE * BLK)

  def pass_copies(p, parity):
    r, co, vr_, vi_, sem = st[parity]
    off = pl.multiple_of(col0 + p * SWP, BLK)
    return [
        pltpu.make_async_copy(rows_hbm.at[:, pl.ds(off, SWP)], r, sem),
        pltpu.make_async_copy(cols_hbm.at[:, pl.ds(off, SWP)], co, sem),
        pltpu.make_async_copy(vr_hbm.at[:, pl.ds(off, SWP)], vr_, sem),
        pltpu.make_async_copy(vi_hbm.at[:, pl.ds(off, SWP)], vi_, sem),
    ]

  def fire_pass(p, parity):
    for cp in pass_copies(p, parity):
      cp.start()

  def wait_pass(p, parity):
    for cp in pass_copies(p, parity):
      cp.wait()

  fire_pass(0, 0)

  @plsc.parallel_loop(0, NP // L, unroll=5)
  def _(k):
    off = pl.ds(k * L, L)
    accr0[off] = zeros
    acci0[off] = zeros
    accr1[off] = zeros
    acci1[off] = zeros

  with jax.named_scope("x_wait"):
    for cp in x_copies:
      cp.wait()

  def phase1_block(r_ref, c_ref, vr_ref, vi_ref, nvregs):
    for b_local in range(2):
      brow16 = jnp.full((L,), 2 * c + b_local, jnp.int32)
      acc_r, acc_i = accs[b_local]
      xbase = b_local * NP

      @plsc.parallel_loop(0, nvregs, unroll=4)
      def _(t):
        idx16 = t * L + iota
        rowg = plsc.load_gather(r_ref, [brow16, idx16])
        colg = plsc.load_gather(c_ref, [brow16, idx16])
        wr = plsc.load_gather(vr_ref, [brow16, idx16])
        wi = plsc.load_gather(vi_ref, [brow16, idx16])
        xcr = plsc.load_gather(xr2, [colg + xbase])
        xci = plsc.load_gather(xi2, [colg + xbase])
        ar = wr * xcr - wi * xci
        ai = wr * xci + wi * xcr
        plsc.addupdate_scatter(acc_r, [rowg], ar)
        plsc.addupdate_scatter(acc_i, [rowg], ai)

  def pass_pair(k, _):
    p0 = k * 2
    fire_pass(p0 + 1, 1)
    with jax.named_scope("st_wait"):
      wait_pass(p0, 0)
    with jax.named_scope("phase1"):
      phase1_block(str0, stc0, stvr0, stvi0, SWP // L)

    @pl.when(p0 + 2 < NPASS)
    def _():
      fire_pass(p0 + 2, 0)

    with jax.named_scope("st_wait"):
      wait_pass(p0 + 1, 1)
    with jax.named_scope("phase1"):
      phase1_block(str1, stc1, stvr1, stvi1, SWP // L)
    return 0

  lax.fori_loop(0, NPASS // 2, pass_pair, 0)

  @pl.when(s == 0)
  def _():
    pltpu.sync_copy(rows_hbm.at[:, pl.ds(TAIL_OFF, TAIL_W)],
                    str0.at[:, pl.ds(0, TAIL_W)])
    pltpu.sync_copy(cols_hbm.at[:, pl.ds(TAIL_OFF, TAIL_W)],
                    stc0.at[:, pl.ds(0, TAIL_W)])
    pltpu.sync_copy(vr_hbm.at[:, pl.ds(TAIL_OFF, TAIL_W)],
                    stvr0.at[:, pl.ds(0, TAIL_W)])
    pltpu.sync_copy(vi_hbm.at[:, pl.ds(TAIL_OFF, TAIL_W)],
                    stvi0.at[:, pl.ds(0, TAIL_W)])
    phase1_block(str0, stc0, stvr0, stvi0, TAIL_W // L)

  row_base = s * SLICE
  b_off = pl.multiple_of(jnp.minimum(row_base, NP - SLICE), 8)

  def b_slices(bi):
    boff = pl.multiple_of(bi * NP, 8) + b_off
    return (br_hbm.at[pl.ds(boff, SLICE)], bi_hbm.at[pl.ds(boff, SLICE)])

  def pub_off(tile, slot):
    return pl.multiple_of(((c * NS + tile) * 4 + slot) * NP, 8)

  with jax.named_scope("publish"):
    pubs = [
        pltpu.async_copy(accr0, exch_hbm.at[pl.ds(pub_off(s, 0), NP)], sem_b),
        pltpu.async_copy(acci0, exch_hbm.at[pl.ds(pub_off(s, 1), NP)], sem_b),
        pltpu.async_copy(accr1, exch_hbm.at[pl.ds(pub_off(s, 2), NP)], sem_b),
        pltpu.async_copy(acci1, exch_hbm.at[pl.ds(pub_off(s, 3), NP)], sem_b),
    ]
    for cp in pubs:
      cp.wait()
    plsc.subcore_barrier()

  b_shift = row_base - b_off
  loss_vec = zeros
  for b_local in range(2):
    bi = 2 * c + b_local
    src_r, src_i = b_slices(bi)
    with jax.named_scope("b_copy"):
      bcp = [pltpu.async_copy(src_r, brv, sem_b),
             pltpu.async_copy(src_i, biv, sem_b)]
    for comp in range(2):
      slot = 2 * b_local + comp
      with jax.named_scope("tmp_copy"):
        tcp = [
            pltpu.async_copy(
                exch_hbm.at[pl.ds(
                    pl.multiple_of(pub_off(t, slot) + b_off, 8), SLICE)],
                tmp.at[pl.ds(t * SLICE, SLICE)], sem_t)
            for t in range(NS)
        ]
        for cp in tcp:
          cp.wait()
      if comp == 0:
        for cp in bcp:
          cp.wait()
      b_ref = brv if comp == 0 else biv

      @plsc.parallel_loop(0, SLICE // L, carry=loss_vec)
      def res_body(k, acc):
        koff = jnp.minimum(b_shift + k * L, SLICE - L)
        ax = tmp[pl.ds(koff, L)]
        for t in range(1, NS):
          ax = ax + tmp[pl.ds(t * SLICE + koff, L)]
        r = b_ref[pl.ds(koff, L)] - ax
        row_vec = row_base + k * L + iota
        return acc + jnp.where(row_vec < NP, r * r, jnp.float32(0.0))
      loss_vec = res_body

  mse_shift = wid * MSE_CHUNK - mse_off
  with jax.named_scope("mse_wait"):
    for cp in mse_copies:
      cp.wait()

  @plsc.parallel_loop(0, MSE_CHUNK // L, carry=loss_vec)
  def mse_body(k, acc):
    roff = jnp.minimum(mse_shift + k * L, MSE_CHUNK - L)
    off = pl.ds(roff, L)
    idx16 = roff + iota
    yr = plsc.load_gather(myf, [2 * idx16])
    yi = plsc.load_gather(myf, [2 * idx16 + 1])
    dr = m0[off] - yr
    di = m1[off] - yi
    contrib = dr * dr + di * di
    elem = wid * MSE_CHUNK + k * L + iota
    return acc + jnp.where(elem < N, contrib, jnp.float32(0.0))
  loss_vec = mse_body

  outv[...] = loss_vec * jnp.float32(0.5 / N)
  pltpu.sync_copy(outv, out_hbm.at[wid])


@jax.jit
def _run(er, ei, y, rows, cols, vr, vi, br, bi):
  mesh = plsc.VectorSubcoreMesh(
      core_axis_name="c", subcore_axis_name="s",
      num_cores=NC, num_subcores=NS)
  f = pl.kernel(
      _sc_body,
      out_type=jax.ShapeDtypeStruct((NC * NS, L), jnp.float32),
      mesh=mesh,
      compiler_params=pltpu.CompilerParams(needs_layout_passes=False),
      scratch_types=[
          pltpu.HBM((NC * NS * 4 * NP,), jnp.float32),
          pltpu.VMEM((B, SWP), jnp.int32),
          pltpu.VMEM((B, SWP), jnp.int32),
          pltpu.VMEM((B, SWP), jnp.float32),
          pltpu.VMEM((B, SWP), jnp.float32),
          pltpu.VMEM((B, SWP), jnp.int32),
          pltpu.VMEM((B, SWP), jnp.int32),
          pltpu.VMEM((B, SWP), jnp.float32),
          pltpu.VMEM((B, SWP), jnp.float32),
          pltpu.VMEM((NP,), jnp.float32),
          pltpu.VMEM((NP,), jnp.float32),
          pltpu.VMEM((NP,), jnp.float32),
          pltpu.VMEM((NP,), jnp.float32),
          pltpu.VMEM((2 * NP,), jnp.float32),
          pltpu.VMEM((2 * NP,), jnp.float32),
          pltpu.VMEM((NS * SLICE,), jnp.float32),
          pltpu.VMEM((MSE_CHUNK,), jnp.float32),
          pltpu.VMEM((MSE_CHUNK,), jnp.float32),
          pltpu.VMEM((2 * MSE_CHUNK,), jnp.float32),
          pltpu.VMEM((SLICE,), jnp.float32),
          pltpu.VMEM((SLICE,), jnp.float32),
          pltpu.VMEM((L,), jnp.float32),
          pltpu.SemaphoreType.DMA,
          pltpu.SemaphoreType.DMA,
          pltpu.SemaphoreType.DMA,
          pltpu.SemaphoreType.DMA,
          pltpu.SemaphoreType.DMA,
          pltpu.SemaphoreType.DMA,
      ],
  )
  return f(er, ei, y, rows, cols, vr, vi, br, bi)


def kernel(E_real, E_imag, batch_y, k_all, node_batch, A_rows, A_cols,
           A_vals_real, A_vals_imag, b_real, b_imag):
  del k_all, node_batch
  partials = _run(E_real, E_imag, batch_y.reshape(-1), A_rows, A_cols,
                  A_vals_real, A_vals_imag,
                  b_real.reshape(-1), b_imag.reshape(-1))
  return jnp.sum(partials)


# speedup vs baseline: 1.4092x; 1.3766x over previous
import functools

import jax
import jax.numpy as jnp
from jax import lax
from jax.experimental import pallas as pl
from jax.experimental.pallas import tpu as pltpu
from jax.experimental.pallas import tpu_sc as plsc

B = 4
NP = 10000
NNZ = 160000
N = B * NP

NC = 2
NS = 16
L = 16

BLK = 128
NB = NNZ // BLK
NB_TILE = 78
KP = 3
SWP = KP * BLK
NPASS = 26
TAIL_OFF = NS * NB_TILE * BLK
TAIL_W = NNZ - TAIL_OFF
NP_PAD = 10240
SLICE = NP_PAD // NS
MSE_CHUNK = 1280


def _sc_body(er_hbm, ei_hbm, ymr_hbm, ymi_hbm, rows_hbm, cols_hbm, vr_hbm, vi_hbm,
             br_hbm, bi_hbm,
             out_hbm,
             exch_hbm,
             str0, stc0, stvr0, stvi0, str1, stc1, stvr1, stvi1,
             accr0, acci0, accr1, acci1, xr2, xi2,
             tmp, m0, m1, m2, m3, brv, biv, outv,
             sem_st0, sem_st1, sem_x, sem_b, sem_mse, sem_t):
  c = lax.axis_index("c")
  s = lax.axis_index("s")

  zeros = jnp.zeros((L,), jnp.float32)
  iota = lax.broadcasted_iota(jnp.int32, (L,), 0)
  wid = c * NS + s

  st = [(str0, stc0, stvr0, stvi0, sem_st0),
        (str1, stc1, stvr1, stvi1, sem_st1)]
  accs = [(accr0, acci0), (accr1, acci1)]

  x_copies = [
      pltpu.async_copy(er_hbm.at[pl.ds(c * 2 * NP, 2 * NP)], xr2, sem_x),
      pltpu.async_copy(ei_hbm.at[pl.ds(c * 2 * NP, 2 * NP)], xi2, sem_x),
  ]
  mse_off = pl.multiple_of(jnp.minimum(wid * MSE_CHUNK, N - MSE_CHUNK), 8)
  mse_copies = [
      pltpu.async_copy(er_hbm.at[pl.ds(mse_off, MSE_CHUNK)], m0, sem_mse),
      pltpu.async_copy(ei_hbm.at[pl.ds(mse_off, MSE_CHUNK)], m1, sem_mse),
      pltpu.async_copy(ymr_hbm.at[pl.ds(mse_off, MSE_CHUNK)], m2, sem_mse),
      pltpu.async_copy(ymi_hbm.at[pl.ds(mse_off, MSE_CHUNK)], m3, sem_mse),
  ]

  col0 = s * (NB_TILE * BLK)

  def pass_copies(p, parity):
    r, co, vr_, vi_, sem = st[parity]
    off = pl.multiple_of(col0 + p * SWP, BLK)
    return [
        pltpu.make_async_copy(rows_hbm.at[:, pl.ds(off, SWP)], r, sem),
        pltpu.make_async_copy(cols_hbm.at[:, pl.ds(off, SWP)], co, sem),
        pltpu.make_async_copy(vr_hbm.at[:, pl.ds(off, SWP)], vr_, sem),
        pltpu.make_async_copy(vi_hbm.at[:, pl.ds(off, SWP)], vi_, sem),
    ]

  def fire_pass(p, parity):
    for cp in pass_copies(p, parity):
      cp.start()

  def wait_pass(p, parity):
    for cp in pass_copies(p, parity):
      cp.wait()

  fire_pass(0, 0)

  @plsc.parallel_loop(0, NP // L, unroll=5)
  def _(k):
    off = pl.ds(k * L, L)
    accr0[off] = zeros
    acci0[off] = zeros
    accr1[off] = zeros
    acci1[off] = zeros

  with jax.named_scope("x_wait"):
    for cp in x_copies:
      cp.wait()

  def phase1_block(r_ref, c_ref, vr_ref, vi_ref, nvregs):
    for b_local in range(2):
      brow16 = jnp.full((L,), 2 * c + b_local, jnp.int32)
      acc_r, acc_i = accs[b_local]
      xbase = b_local * NP

      @plsc.parallel_loop(0, nvregs, unroll=4)
      def _(t):
        idx16 = t * L + iota
        rowg = plsc.load_gather(r_ref, [brow16, idx16])
        colg = plsc.load_gather(c_ref, [brow16, idx16])
        wr = plsc.load_gather(vr_ref, [brow16, idx16])
        wi = plsc.load_gather(vi_ref, [brow16, idx16])
        xcr = plsc.load_gather(xr2, [colg + xbase])
        xci = plsc.load_gather(xi2, [colg + xbase])
        ar = wr * xcr - wi * xci
        ai = wr * xci + wi * xcr
        plsc.addupdate_scatter(acc_r, [rowg], ar)
        plsc.addupdate_scatter(acc_i, [rowg], ai)

  def pass_pair(k, _):
    p0 = k * 2
    fire_pass(p0 + 1, 1)
    with jax.named_scope("st_wait"):
      wait_pass(p0, 0)
    with jax.named_scope("phase1"):
      phase1_block(str0, stc0, stvr0, stvi0, SWP // L)

    @pl.when(p0 + 2 < NPASS)
    def _():
      fire_pass(p0 + 2, 0)

    with jax.named_scope("st_wait"):
      wait_pass(p0 + 1, 1)
    with jax.named_scope("phase1"):
      phase1_block(str1, stc1, stvr1, stvi1, SWP // L)
    return 0

  lax.fori_loop(0, NPASS // 2, pass_pair, 0)

  @pl.when(s == 0)
  def _():
    pltpu.sync_copy(rows_hbm.at[:, pl.ds(TAIL_OFF, TAIL_W)],
                    str0.at[:, pl.ds(0, TAIL_W)])
    pltpu.sync_copy(cols_hbm.at[:, pl.ds(TAIL_OFF, TAIL_W)],
                    stc0.at[:, pl.ds(0, TAIL_W)])
    pltpu.sync_copy(vr_hbm.at[:, pl.ds(TAIL_OFF, TAIL_W)],
                    stvr0.at[:, pl.ds(0, TAIL_W)])
    pltpu.sync_copy(vi_hbm.at[:, pl.ds(TAIL_OFF, TAIL_W)],
                    stvi0.at[:, pl.ds(0, TAIL_W)])
    phase1_block(str0, stc0, stvr0, stvi0, TAIL_W // L)

  row_base = s * SLICE
  b_off = pl.multiple_of(jnp.minimum(row_base, NP - SLICE), 8)

  def b_slices(bi):
    boff = pl.multiple_of(bi * NP, 8) + b_off
    return (br_hbm.at[pl.ds(boff, SLICE)], bi_hbm.at[pl.ds(boff, SLICE)])

  def pub_off(tile, slot):
    return pl.multiple_of(((c * NS + tile) * 4 + slot) * NP, 8)

  with jax.named_scope("publish"):
    pubs = [
        pltpu.async_copy(accr0, exch_hbm.at[pl.ds(pub_off(s, 0), NP)], sem_b),
        pltpu.async_copy(acci0, exch_hbm.at[pl.ds(pub_off(s, 1), NP)], sem_b),
        pltpu.async_copy(accr1, exch_hbm.at[pl.ds(pub_off(s, 2), NP)], sem_b),
        pltpu.async_copy(acci1, exch_hbm.at[pl.ds(pub_off(s, 3), NP)], sem_b),
    ]
    for cp in pubs:
      cp.wait()
    plsc.subcore_barrier()

  b_shift = row_base - b_off
  loss_vec = zeros
  for b_local in range(2):
    bi = 2 * c + b_local
    src_r, src_i = b_slices(bi)
    with jax.named_scope("b_copy"):
      bcp = [pltpu.async_copy(src_r, brv, sem_b),
             pltpu.async_copy(src_i, biv, sem_b)]
    for comp in range(2):
      slot = 2 * b_local + comp
      with jax.named_scope("tmp_copy"):
        tcp = [
            pltpu.async_copy(
                exch_hbm.at[pl.ds(
                    pl.multiple_of(pub_off(t, slot) + b_off, 8), SLICE)],
                tmp.at[pl.ds(t * SLICE, SLICE)], sem_t)
            for t in range(NS)
        ]
        for cp in tcp:
          cp.wait()
      if comp == 0:
        for cp in bcp:
          cp.wait()
      b_ref = brv if comp == 0 else biv

      @plsc.parallel_loop(0, SLICE // L, carry=loss_vec)
      def res_body(k, acc):
        koff = jnp.minimum(b_shift + k * L, SLICE - L)
        ax = tmp[pl.ds(koff, L)]
        for t in range(1, NS):
          ax = ax + tmp[pl.ds(t * SLICE + koff, L)]
        r = b_ref[pl.ds(koff, L)] - ax
        row_vec = row_base + k * L + iota
        return acc + jnp.where(row_vec < NP, r * r, jnp.float32(0.0))
      loss_vec = res_body

  mse_shift = wid * MSE_CHUNK - mse_off
  with jax.named_scope("mse_wait"):
    for cp in mse_copies:
      cp.wait()

  @plsc.parallel_loop(0, MSE_CHUNK // L, carry=loss_vec)
  def mse_body(k, acc):
    roff = jnp.minimum(mse_shift + k * L, MSE_CHUNK - L)
    off = pl.ds(roff, L)
    dr = m0[off] - m2[off]
    di = m1[off] - m3[off]
    contrib = dr * dr + di * di
    elem = wid * MSE_CHUNK + k * L + iota
    return acc + jnp.where(elem < N, contrib, jnp.float32(0.0))
  loss_vec = mse_body

  outv[...] = loss_vec * jnp.float32(0.5 / N)
  pltpu.sync_copy(outv, out_hbm.at[wid])


@jax.jit
def _run(er, ei, ymr, ymi, rows, cols, vr, vi, br, bi):
  mesh = plsc.VectorSubcoreMesh(
      core_axis_name="c", subcore_axis_name="s",
      num_cores=NC, num_subcores=NS)
  f = pl.kernel(
      _sc_body,
      out_type=jax.ShapeDtypeStruct((NC * NS, L), jnp.float32),
      mesh=mesh,
      compiler_params=pltpu.CompilerParams(needs_layout_passes=False),
      scratch_types=[
          pltpu.HBM((NC * NS * 4 * NP,), jnp.float32),
          pltpu.VMEM((B, SWP), jnp.int32),
          pltpu.VMEM((B, SWP), jnp.int32),
          pltpu.VMEM((B, SWP), jnp.float32),
          pltpu.VMEM((B, SWP), jnp.float32),
          pltpu.VMEM((B, SWP), jnp.int32),
          pltpu.VMEM((B, SWP), jnp.int32),
          pltpu.VMEM((B, SWP), jnp.float32),
          pltpu.VMEM((B, SWP), jnp.float32),
          pltpu.VMEM((NP,), jnp.float32),
          pltpu.VMEM((NP,), jnp.float32),
          pltpu.VMEM((NP,), jnp.float32),
          pltpu.VMEM((NP,), jnp.float32),
          pltpu.VMEM((2 * NP,), jnp.float32),
          pltpu.VMEM((2 * NP,), jnp.float32),
          pltpu.VMEM((NS * SLICE,), jnp.float32),
          pltpu.VMEM((MSE_CHUNK,), jnp.float32),
          pltpu.VMEM((MSE_CHUNK,), jnp.float32),
          pltpu.VMEM((MSE_CHUNK,), jnp.float32),
          pltpu.VMEM((MSE_CHUNK,), jnp.float32),
          pltpu.VMEM((SLICE,), jnp.float32),
          pltpu.VMEM((SLICE,), jnp.float32),
          pltpu.VMEM((L,), jnp.float32),
          pltpu.SemaphoreType.DMA,
          pltpu.SemaphoreType.DMA,
          pltpu.SemaphoreType.DMA,
          pltpu.SemaphoreType.DMA,
          pltpu.SemaphoreType.DMA,
          pltpu.SemaphoreType.DMA,
      ],
  )
  return f(er, ei, ymr, ymi, rows, cols, vr, vi, br, bi)


def kernel(E_real, E_imag, batch_y, k_all, node_batch, A_rows, A_cols,
           A_vals_real, A_vals_imag, b_real, b_imag):
  del k_all, node_batch
  partials = _run(E_real, E_imag, batch_y[:, 0], batch_y[:, 1],
                  A_rows, A_cols, A_vals_real, A_vals_imag,
                  b_real.reshape(-1), b_imag.reshape(-1))
  return jnp.sum(partials)


# speedup vs baseline: 1.4527x; 1.0309x over previous
import functools

import jax
import jax.numpy as jnp
from jax import lax
from jax.experimental import pallas as pl
from jax.experimental.pallas import tpu as pltpu
from jax.experimental.pallas import tpu_sc as plsc

B = 4
NP = 10000
NNZ = 160000
N = B * NP

NC = 2
NS = 16
L = 16

BLK = 128
NB = NNZ // BLK
NB_TILE = 78
KP = 2
SWP = KP * BLK
NPASS = 39
TAIL_OFF = NS * NB_TILE * BLK
TAIL_W = NNZ - TAIL_OFF
NP_PAD = 10240
SLICE = NP_PAD // NS
MSE_CHUNK = 1280


def _sc_body(er_hbm, ei_hbm, ymr_hbm, ymi_hbm, rows_hbm, cols_hbm, vr_hbm, vi_hbm,
             br_hbm, bi_hbm,
             out_hbm,
             exch_hbm,
             str0, stc0, stvr0, stvi0, str1, stc1, stvr1, stvi1,
             str2, stc2, stvr2, stvi2,
             accr0, acci0, accr1, acci1, xr2, xi2,
             tmp, m0, m1, m2, m3, brv, biv, outv,
             sem_st0, sem_st1, sem_st2, sem_x, sem_b, sem_mse, sem_t):
  c = lax.axis_index("c")
  s = lax.axis_index("s")

  zeros = jnp.zeros((L,), jnp.float32)
  iota = lax.broadcasted_iota(jnp.int32, (L,), 0)
  wid = c * NS + s

  st = [(str0, stc0, stvr0, stvi0, sem_st0),
        (str1, stc1, stvr1, stvi1, sem_st1),
        (str2, stc2, stvr2, stvi2, sem_st2)]
  accs = [(accr0, acci0), (accr1, acci1)]

  x_copies = [
      pltpu.async_copy(er_hbm.at[pl.ds(c * 2 * NP, 2 * NP)], xr2, sem_x),
      pltpu.async_copy(ei_hbm.at[pl.ds(c * 2 * NP, 2 * NP)], xi2, sem_x),
  ]
  mse_off = pl.multiple_of(jnp.minimum(wid * MSE_CHUNK, N - MSE_CHUNK), 8)
  mse_copies = [
      pltpu.async_copy(er_hbm.at[pl.ds(mse_off, MSE_CHUNK)], m0, sem_mse),
      pltpu.async_copy(ei_hbm.at[pl.ds(mse_off, MSE_CHUNK)], m1, sem_mse),
      pltpu.async_copy(ymr_hbm.at[pl.ds(mse_off, MSE_CHUNK)], m2, sem_mse),
      pltpu.async_copy(ymi_hbm.at[pl.ds(mse_off, MSE_CHUNK)], m3, sem_mse),
  ]

  col0 = s * (NB_TILE * BLK)

  def pass_copies(p, parity):
    r, co, vr_, vi_, sem = st[parity]
    off = pl.multiple_of(col0 + p * SWP, BLK)
    return [
        pltpu.make_async_copy(rows_hbm.at[:, pl.ds(off, SWP)], r, sem),
        pltpu.make_async_copy(cols_hbm.at[:, pl.ds(off, SWP)], co, sem),
        pltpu.make_async_copy(vr_hbm.at[:, pl.ds(off, SWP)], vr_, sem),
        pltpu.make_async_copy(vi_hbm.at[:, pl.ds(off, SWP)], vi_, sem),
    ]

  def fire_pass(p, parity):
    for cp in pass_copies(p, parity):
      cp.start()

  def wait_pass(p, parity):
    for cp in pass_copies(p, parity):
      cp.wait()

  fire_pass(0, 0)
  fire_pass(1, 1)

  @plsc.parallel_loop(0, NP // L, unroll=5)
  def _(k):
    off = pl.ds(k * L, L)
    accr0[off] = zeros
    acci0[off] = zeros
    accr1[off] = zeros
    acci1[off] = zeros

  with jax.named_scope("x_wait"):
    for cp in x_copies:
      cp.wait()

  def phase1_block(r_ref, c_ref, vr_ref, vi_ref, nvregs):
    for b_local in range(2):
      brow16 = jnp.full((L,), 2 * c + b_local, jnp.int32)
      acc_r, acc_i = accs[b_local]
      xbase = b_local * NP

      @plsc.parallel_loop(0, nvregs, unroll=8)
      def _(t):
        idx16 = t * L + iota
        rowg = plsc.load_gather(r_ref, [brow16, idx16])
        colg = plsc.load_gather(c_ref, [brow16, idx16])
        wr = plsc.load_gather(vr_ref, [brow16, idx16])
        wi = plsc.load_gather(vi_ref, [brow16, idx16])
        xcr = plsc.load_gather(xr2, [colg + xbase])
        xci = plsc.load_gather(xi2, [colg + xbase])
        ar = wr * xcr - wi * xci
        ai = wr * xci + wi * xcr
        plsc.addupdate_scatter(acc_r, [rowg], ar)
        plsc.addupdate_scatter(acc_i, [rowg], ai)

  def pass_triple(k, _):
    p0 = k * 3
    for j in range(3):
      p = p0 + j

      @pl.when(p + 2 < NPASS)
      def _():
        fire_pass(p + 2, (j + 2) % 3)

      with jax.named_scope("st_wait"):
        wait_pass(p, j)
      r_ref, c_ref, vr_ref, vi_ref, _ = st[j]
      with jax.named_scope("phase1"):
        phase1_block(r_ref, c_ref, vr_ref, vi_ref, SWP // L)
    return 0

  lax.fori_loop(0, NPASS // 3, pass_triple, 0)

  @pl.when(s == 0)
  def _():
    pltpu.sync_copy(rows_hbm.at[:, pl.ds(TAIL_OFF, TAIL_W)],
                    str0.at[:, pl.ds(0, TAIL_W)])
    pltpu.sync_copy(cols_hbm.at[:, pl.ds(TAIL_OFF, TAIL_W)],
                    stc0.at[:, pl.ds(0, TAIL_W)])
    pltpu.sync_copy(vr_hbm.at[:, pl.ds(TAIL_OFF, TAIL_W)],
                    stvr0.at[:, pl.ds(0, TAIL_W)])
    pltpu.sync_copy(vi_hbm.at[:, pl.ds(TAIL_OFF, TAIL_W)],
                    stvi0.at[:, pl.ds(0, TAIL_W)])
    phase1_block(str0, stc0, stvr0, stvi0, TAIL_W // L)

  row_base = s * SLICE
  b_off = pl.multiple_of(jnp.minimum(row_base, NP - SLICE), 8)

  def b_slices(bi):
    boff = pl.multiple_of(bi * NP, 8) + b_off
    return (br_hbm.at[pl.ds(boff, SLICE)], bi_hbm.at[pl.ds(boff, SLICE)])

  def pub_off(tile, slot):
    return pl.multiple_of(((c * NS + tile) * 4 + slot) * NP, 8)

  with jax.named_scope("publish"):
    pubs = [
        pltpu.async_copy(accr0, exch_hbm.at[pl.ds(pub_off(s, 0), NP)], sem_b),
        pltpu.async_copy(acci0, exch_hbm.at[pl.ds(pub_off(s, 1), NP)], sem_b),
        pltpu.async_copy(accr1, exch_hbm.at[pl.ds(pub_off(s, 2), NP)], sem_b),
        pltpu.async_copy(acci1, exch_hbm.at[pl.ds(pub_off(s, 3), NP)], sem_b),
    ]
    for cp in pubs:
      cp.wait()
    plsc.subcore_barrier()

  b_shift = row_base - b_off
  loss_vec = zeros
  for b_local in range(2):
    bi = 2 * c + b_local
    src_r, src_i = b_slices(bi)
    with jax.named_scope("b_copy"):
      bcp = [pltpu.async_copy(src_r, brv, sem_b),
             pltpu.async_copy(src_i, biv, sem_b)]
    for comp in range(2):
      slot = 2 * b_local + comp
      with jax.named_scope("tmp_copy"):
        tcp = [
            pltpu.async_copy(
                exch_hbm.at[pl.ds(
                    pl.multiple_of(pub_off(t, slot) + b_off, 8), SLICE)],
                tmp.at[pl.ds(t * SLICE, SLICE)], sem_t)
            for t in range(NS)
        ]
        for cp in tcp:
          cp.wait()
      if comp == 0:
        for cp in bcp:
          cp.wait()
      b_ref = brv if comp == 0 else biv

      @plsc.parallel_loop(0, SLICE // L, carry=loss_vec)
      def res_body(k, acc):
        koff = jnp.minimum(b_shift + k * L, SLICE - L)
        ax = tmp[pl.ds(koff, L)]
        for t in range(1, NS):
          ax = ax + tmp[pl.ds(t * SLICE + koff, L)]
        r = b_ref[pl.ds(koff, L)] - ax
        row_vec = row_base + k * L + iota
        return acc + jnp.where(row_vec < NP, r * r, jnp.float32(0.0))
      loss_vec = res_body

  mse_shift = wid * MSE_CHUNK - mse_off
  with jax.named_scope("mse_wait"):
    for cp in mse_copies:
      cp.wait()

  @plsc.parallel_loop(0, MSE_CHUNK // L, carry=loss_vec)
  def mse_body(k, acc):
    roff = jnp.minimum(mse_shift + k * L, MSE_CHUNK - L)
    off = pl.ds(roff, L)
    dr = m0[off] - m2[off]
    di = m1[off] - m3[off]
    contrib = dr * dr + di * di
    elem = wid * MSE_CHUNK + k * L + iota
    return acc + jnp.where(elem < N, contrib, jnp.float32(0.0))
  loss_vec = mse_body

  outv[...] = loss_vec * jnp.float32(0.5 / N)
  pltpu.sync_copy(outv, out_hbm.at[wid])


@jax.jit
def _run(er, ei, ymr, ymi, rows, cols, vr, vi, br, bi):
  mesh = plsc.VectorSubcoreMesh(
      core_axis_name="c", subcore_axis_name="s",
      num_cores=NC, num_subcores=NS)
  f = pl.kernel(
      _sc_body,
      out_type=jax.ShapeDtypeStruct((NC * NS, L), jnp.float32),
      mesh=mesh,
      compiler_params=pltpu.CompilerParams(needs_layout_passes=False),
      scratch_types=[
          pltpu.HBM((NC * NS * 4 * NP,), jnp.float32),
          pltpu.VMEM((B, SWP), jnp.int32),
          pltpu.VMEM((B, SWP), jnp.int32),
          pltpu.VMEM((B, SWP), jnp.float32),
          pltpu.VMEM((B, SWP), jnp.float32),
          pltpu.VMEM((B, SWP), jnp.int32),
          pltpu.VMEM((B, SWP), jnp.int32),
          pltpu.VMEM((B, SWP), jnp.float32),
          pltpu.VMEM((B, SWP), jnp.float32),
          pltpu.VMEM((B, SWP), jnp.int32),
          pltpu.VMEM((B, SWP), jnp.int32),
          pltpu.VMEM((B, SWP), jnp.float32),
          pltpu.VMEM((B, SWP), jnp.float32),
          pltpu.VMEM((NP,), jnp.float32),
          pltpu.VMEM((NP,), jnp.float32),
          pltpu.VMEM((NP,), jnp.float32),
          pltpu.VMEM((NP,), jnp.float32),
          pltpu.VMEM((2 * NP,), jnp.float32),
          pltpu.VMEM((2 * NP,), jnp.float32),
          pltpu.VMEM((NS * SLICE,), jnp.float32),
          pltpu.VMEM((MSE_CHUNK,), jnp.float32),
          pltpu.VMEM((MSE_CHUNK,), jnp.float32),
          pltpu.VMEM((MSE_CHUNK,), jnp.float32),
          pltpu.VMEM((MSE_CHUNK,), jnp.float32),
          pltpu.VMEM((SLICE,), jnp.float32),
          pltpu.VMEM((SLICE,), jnp.float32),
          pltpu.VMEM((L,), jnp.float32),
          pltpu.SemaphoreType.DMA,
          pltpu.SemaphoreType.DMA,
          pltpu.SemaphoreType.DMA,
          pltpu.SemaphoreType.DMA,
          pltpu.SemaphoreType.DMA,
          pltpu.SemaphoreType.DMA,
          pltpu.SemaphoreType.DMA,
      ],
  )
  return f(er, ei, ymr, ymi, rows, cols, vr, vi, br, bi)


def kernel(E_real, E_imag, batch_y, k_all, node_batch, A_rows, A_cols,
           A_vals_real, A_vals_imag, b_real, b_imag):
  del k_all, node_batch
  partials = _run(E_real, E_imag, batch_y[:, 0], batch_y[:, 1],
                  A_rows, A_cols, A_vals_real, A_vals_imag,
                  b_real.reshape(-1), b_imag.reshape(-1))
  return jnp.sum(partials)


# speedup vs baseline: 1.4706x; 1.0123x over previous
import functools

import jax
import jax.numpy as jnp
from jax import lax
from jax.experimental import pallas as pl
from jax.experimental.pallas import tpu as pltpu
from jax.experimental.pallas import tpu_sc as plsc

B = 4
NP = 10000
NNZ = 160000
N = B * NP

NC = 2
NS = 16
L = 16

BLK = 128
NB = NNZ // BLK
NB_TILE = 78
KP = 2
SWP = KP * BLK
NPASS = 39
TAIL_OFF = NS * NB_TILE * BLK
TAIL_W = NNZ - TAIL_OFF
NP_PAD = 10240
SLICE = NP_PAD // NS
MSE_CHUNK = 1280


def _sc_body(er_hbm, ei_hbm, ymr_hbm, ymi_hbm, rows_hbm, cols_hbm, vr_hbm, vi_hbm,
             br_hbm, bi_hbm,
             out_hbm,
             exch_hbm,
             str0, stc0, stvr0, stvi0, str1, stc1, stvr1, stvi1,
             str2, stc2, stvr2, stvi2,
             accr0, acci0, accr1, acci1, xr2, xi2,
             tmpa, tmpb, axbuf, m0, m1, m2, m3, brv, biv, outv,
             sem_st0, sem_st1, sem_st2, sem_x, sem_b, sem_mse, sem_t,
             sem_t2):
  c = lax.axis_index("c")
  s = lax.axis_index("s")

  zeros = jnp.zeros((L,), jnp.float32)
  iota = lax.broadcasted_iota(jnp.int32, (L,), 0)
  wid = c * NS + s

  st = [(str0, stc0, stvr0, stvi0, sem_st0),
        (str1, stc1, stvr1, stvi1, sem_st1),
        (str2, stc2, stvr2, stvi2, sem_st2)]
  accs = [(accr0, acci0), (accr1, acci1)]

  x_copies = [
      pltpu.async_copy(er_hbm.at[pl.ds(c * 2 * NP, 2 * NP)], xr2, sem_x),
      pltpu.async_copy(ei_hbm.at[pl.ds(c * 2 * NP, 2 * NP)], xi2, sem_x),
  ]
  mse_off = pl.multiple_of(jnp.minimum(wid * MSE_CHUNK, N - MSE_CHUNK), 8)
  mse_copies = [
      pltpu.async_copy(er_hbm.at[pl.ds(mse_off, MSE_CHUNK)], m0, sem_mse),
      pltpu.async_copy(ei_hbm.at[pl.ds(mse_off, MSE_CHUNK)], m1, sem_mse),
      pltpu.async_copy(ymr_hbm.at[pl.ds(mse_off, MSE_CHUNK)], m2, sem_mse),
      pltpu.async_copy(ymi_hbm.at[pl.ds(mse_off, MSE_CHUNK)], m3, sem_mse),
  ]

  col0 = s * (NB_TILE * BLK)

  def pass_copies(p, parity):
    r, co, vr_, vi_, sem = st[parity]
    off = pl.multiple_of(col0 + p * SWP, BLK)
    return [
        pltpu.make_async_copy(rows_hbm.at[:, pl.ds(off, SWP)], r, sem),
        pltpu.make_async_copy(cols_hbm.at[:, pl.ds(off, SWP)], co, sem),
        pltpu.make_async_copy(vr_hbm.at[:, pl.ds(off, SWP)], vr_, sem),
        pltpu.make_async_copy(vi_hbm.at[:, pl.ds(off, SWP)], vi_, sem),
    ]

  def fire_pass(p, parity):
    for cp in pass_copies(p, parity):
      cp.start()

  def wait_pass(p, parity):
    for cp in pass_copies(p, parity):
      cp.wait()

  fire_pass(0, 0)
  fire_pass(1, 1)

  @plsc.parallel_loop(0, NP // L, unroll=5)
  def _(k):
    off = pl.ds(k * L, L)
    accr0[off] = zeros
    acci0[off] = zeros
    accr1[off] = zeros
    acci1[off] = zeros

  with jax.named_scope("x_wait"):
    for cp in x_copies:
      cp.wait()

  def phase1_block(r_ref, c_ref, vr_ref, vi_ref, nvregs):
    for b_local in range(2):
      brow16 = jnp.full((L,), 2 * c + b_local, jnp.int32)
      acc_r, acc_i = accs[b_local]
      xbase = b_local * NP

      @plsc.parallel_loop(0, nvregs, unroll=8)
      def _(t):
        idx16 = t * L + iota
        rowg = plsc.load_gather(r_ref, [brow16, idx16])
        colg = plsc.load_gather(c_ref, [brow16, idx16])
        wr = plsc.load_gather(vr_ref, [brow16, idx16])
        wi = plsc.load_gather(vi_ref, [brow16, idx16])
        xcr = plsc.load_gather(xr2, [colg + xbase])
        xci = plsc.load_gather(xi2, [colg + xbase])
        ar = wr * xcr - wi * xci
        ai = wr * xci + wi * xcr
        plsc.addupdate_scatter(acc_r, [rowg], ar)
        plsc.addupdate_scatter(acc_i, [rowg], ai)

  def pass_triple(k, _):
    p0 = k * 3
    for j in range(3):
      p = p0 + j

      @pl.when(p + 2 < NPASS)
      def _():
        fire_pass(p + 2, (j + 2) % 3)

      with jax.named_scope("st_wait"):
        wait_pass(p, j)
      r_ref, c_ref, vr_ref, vi_ref, _ = st[j]
      with jax.named_scope("phase1"):
        phase1_block(r_ref, c_ref, vr_ref, vi_ref, SWP // L)
    return 0

  lax.fori_loop(0, NPASS // 3, pass_triple, 0)

  @pl.when(s == 0)
  def _():
    pltpu.sync_copy(rows_hbm.at[:, pl.ds(TAIL_OFF, TAIL_W)],
                    str0.at[:, pl.ds(0, TAIL_W)])
    pltpu.sync_copy(cols_hbm.at[:, pl.ds(TAIL_OFF, TAIL_W)],
                    stc0.at[:, pl.ds(0, TAIL_W)])
    pltpu.sync_copy(vr_hbm.at[:, pl.ds(TAIL_OFF, TAIL_W)],
                    stvr0.at[:, pl.ds(0, TAIL_W)])
    pltpu.sync_copy(vi_hbm.at[:, pl.ds(TAIL_OFF, TAIL_W)],
                    stvi0.at[:, pl.ds(0, TAIL_W)])
    phase1_block(str0, stc0, stvr0, stvi0, TAIL_W // L)

  row_base = s * SLICE
  b_off = pl.multiple_of(jnp.minimum(row_base, NP - SLICE), 8)

  def b_slices(bi):
    boff = pl.multiple_of(bi * NP, 8) + b_off
    return (br_hbm.at[pl.ds(boff, SLICE)], bi_hbm.at[pl.ds(boff, SLICE)])

  def pub_off(tile, slot):
    return pl.multiple_of(((c * NS + tile) * 4 + slot) * NP, 8)

  with jax.named_scope("publish"):
    pubs = [
        pltpu.async_copy(accr0, exch_hbm.at[pl.ds(pub_off(s, 0), NP)], sem_b),
        pltpu.async_copy(acci0, exch_hbm.at[pl.ds(pub_off(s, 1), NP)], sem_b),
        pltpu.async_copy(accr1, exch_hbm.at[pl.ds(pub_off(s, 2), NP)], sem_b),
        pltpu.async_copy(acci1, exch_hbm.at[pl.ds(pub_off(s, 3), NP)], sem_b),
    ]
    for cp in pubs:
      cp.wait()
    plsc.subcore_barrier()

  b_shift = row_base - b_off
  loss_vec = zeros
  pairs = [(q, h) for q in range(4) for h in range(2)]
  tmps = [tmpa, tmpb]
  semt = [sem_t, sem_t2]

  def pair_copies(i):
    q, h = pairs[i]
    buf = tmps[i % 2]
    return [
        pltpu.make_async_copy(
            exch_hbm.at[pl.ds(
                pl.multiple_of(pub_off(h * 8 + j, q) + b_off, 8), SLICE)],
            buf.at[pl.ds(j * SLICE, SLICE)], semt[i % 2])
        for j in range(8)
    ]

  for cp in pair_copies(0):
    cp.start()
  bcp = [pltpu.async_copy(b_slices(2 * c)[0], brv, sem_b),
         pltpu.async_copy(b_slices(2 * c)[1], biv, sem_b)]

  for i, (q, h) in enumerate(pairs):
    if i + 1 < len(pairs):
      for cp in pair_copies(i + 1):
        cp.start()
    with jax.named_scope("tmp_copy"):
      for cp in pair_copies(i):
        cp.wait()
    buf = tmps[i % 2]
    b_ref = brv if q % 2 == 0 else biv

    if h == 0:
      @plsc.parallel_loop(0, SLICE // L)
      def half0(k):
        koff = jnp.minimum(b_shift + k * L, SLICE - L)
        part = buf[pl.ds(koff, L)]
        for j in range(1, 8):
          part = part + buf[pl.ds(j * SLICE + koff, L)]
        axbuf[pl.ds(k * L, L)] = part
    else:
      if q == 1:
        pass
      if q == 0:
        with jax.named_scope("b_wait"):
          for cp in bcp:
            cp.wait()

      @plsc.parallel_loop(0, SLICE // L, carry=loss_vec)
      def half1(k, acc):
        koff = jnp.minimum(b_shift + k * L, SLICE - L)
        part = buf[pl.ds(koff, L)]
        for j in range(1, 8):
          part = part + buf[pl.ds(j * SLICE + koff, L)]
        ax = axbuf[pl.ds(k * L, L)] + part
        r = b_ref[pl.ds(koff, L)] - ax
        row_vec = row_base + k * L + iota
        return acc + jnp.where(row_vec < NP, r * r, jnp.float32(0.0))
      loss_vec = half1

      if q == 1:
        bcp = [pltpu.async_copy(b_slices(2 * c + 1)[0], brv, sem_b),
               pltpu.async_copy(b_slices(2 * c + 1)[1], biv, sem_b)]
        with jax.named_scope("b_wait"):
          for cp in bcp:
            cp.wait()

  mse_shift = wid * MSE_CHUNK - mse_off
  with jax.named_scope("mse_wait"):
    for cp in mse_copies:
      cp.wait()

  @plsc.parallel_loop(0, MSE_CHUNK // L, carry=loss_vec)
  def mse_body(k, acc):
    roff = jnp.minimum(mse_shift + k * L, MSE_CHUNK - L)
    off = pl.ds(roff, L)
    dr = m0[off] - m2[off]
    di = m1[off] - m3[off]
    contrib = dr * dr + di * di
    elem = wid * MSE_CHUNK + k * L + iota
    return acc + jnp.where(elem < N, contrib, jnp.float32(0.0))
  loss_vec = mse_body

  outv[...] = loss_vec * jnp.float32(0.5 / N)
  pltpu.sync_copy(outv, out_hbm.at[wid])


@jax.jit
def _run(er, ei, ymr, ymi, rows, cols, vr, vi, br, bi):
  mesh = plsc.VectorSubcoreMesh(
      core_axis_name="c", subcore_axis_name="s",
      num_cores=NC, num_subcores=NS)
  f = pl.kernel(
      _sc_body,
      out_type=jax.ShapeDtypeStruct((NC * NS, L), jnp.float32),
      mesh=mesh,
      compiler_params=pltpu.CompilerParams(needs_layout_passes=False),
      scratch_types=[
          pltpu.HBM((NC * NS * 4 * NP,), jnp.float32),
          pltpu.VMEM((B, SWP), jnp.int32),
          pltpu.VMEM((B, SWP), jnp.int32),
          pltpu.VMEM((B, SWP), jnp.float32),
          pltpu.VMEM((B, SWP), jnp.float32),
          pltpu.VMEM((B, SWP), jnp.int32),
          pltpu.VMEM((B, SWP), jnp.int32),
          pltpu.VMEM((B, SWP), jnp.float32),
          pltpu.VMEM((B, SWP), jnp.float32),
          pltpu.VMEM((B, SWP), jnp.int32),
          pltpu.VMEM((B, SWP), jnp.int32),
          pltpu.VMEM((B, SWP), jnp.float32),
          pltpu.VMEM((B, SWP), jnp.float32),
          pltpu.VMEM((NP,), jnp.float32),
          pltpu.VMEM((NP,), jnp.float32),
          pltpu.VMEM((NP,), jnp.float32),
          pltpu.VMEM((NP,), jnp.float32),
          pltpu.VMEM((2 * NP,), jnp.float32),
          pltpu.VMEM((2 * NP,), jnp.float32),
          pltpu.VMEM((8 * SLICE,), jnp.float32),
          pltpu.VMEM((8 * SLICE,), jnp.float32),
          pltpu.VMEM((SLICE,), jnp.float32),
          pltpu.VMEM((MSE_CHUNK,), jnp.float32),
          pltpu.VMEM((MSE_CHUNK,), jnp.float32),
          pltpu.VMEM((MSE_CHUNK,), jnp.float32),
          pltpu.VMEM((MSE_CHUNK,), jnp.float32),
          pltpu.VMEM((SLICE,), jnp.float32),
          pltpu.VMEM((SLICE,), jnp.float32),
          pltpu.VMEM((L,), jnp.float32),
          pltpu.SemaphoreType.DMA,
          pltpu.SemaphoreType.DMA,
          pltpu.SemaphoreType.DMA,
          pltpu.SemaphoreType.DMA,
          pltpu.SemaphoreType.DMA,
          pltpu.SemaphoreType.DMA,
          pltpu.SemaphoreType.DMA,
          pltpu.SemaphoreType.DMA,
      ],
  )
  return f(er, ei, ymr, ymi, rows, cols, vr, vi, br, bi)


def kernel(E_real, E_imag, batch_y, k_all, node_batch, A_rows, A_cols,
           A_vals_real, A_vals_imag, b_real, b_imag):
  del k_all, node_batch
  partials = _run(E_real, E_imag, batch_y[:, 0], batch_y[:, 1],
                  A_rows, A_cols, A_vals_real, A_vals_imag,
                  b_real.reshape(-1), b_imag.reshape(-1))
  return jnp.sum(partials)


# speedup vs baseline: 1.4788x; 1.0056x over previous
import functools

import jax
import jax.numpy as jnp
from jax import lax
from jax.experimental import pallas as pl
from jax.experimental.pallas import tpu as pltpu
from jax.experimental.pallas import tpu_sc as plsc

B = 4
NP = 10000
NNZ = 160000
N = B * NP

NC = 2
NS = 16
L = 16

BLK = 128
NB = NNZ // BLK
NB_TILE = 78
KP = 2
SWP = KP * BLK
NPASS = 39
TAIL_OFF = NS * NB_TILE * BLK
TAIL_W = NNZ - TAIL_OFF
NP_PAD = 10240
SLICE = NP_PAD // NS
MSE_CHUNK = 1280


def _sc_body(er_hbm, ei_hbm, ymr_hbm, ymi_hbm, rows_hbm, cols_hbm, vr_hbm, vi_hbm,
             br_hbm, bi_hbm,
             out_hbm,
             exch_hbm,
             str0, stc0, stvr0, stvi0, str1, stc1, stvr1, stvi1,
             str2, stc2, stvr2, stvi2,
             accr0, acci0, accr1, acci1, xr2, xi2,
             tmpa, tmpb, axbuf, m0, m1, m2, m3, brv, biv, outv,
             sem_st0, sem_st1, sem_st2, sem_x, sem_b, sem_mse, sem_t,
             sem_t2):
  c = lax.axis_index("c")
  s = lax.axis_index("s")

  zeros = jnp.zeros((L,), jnp.float32)
  iota = lax.broadcasted_iota(jnp.int32, (L,), 0)
  wid = c * NS + s

  st = [(str0, stc0, stvr0, stvi0, sem_st0),
        (str1, stc1, stvr1, stvi1, sem_st1),
        (str2, stc2, stvr2, stvi2, sem_st2)]
  accs = [(accr0, acci0), (accr1, acci1)]

  x_copies = [
      pltpu.async_copy(er_hbm.at[pl.ds(c * 2 * NP, 2 * NP)], xr2, sem_x),
      pltpu.async_copy(ei_hbm.at[pl.ds(c * 2 * NP, 2 * NP)], xi2, sem_x),
  ]
  mse_off = pl.multiple_of(jnp.minimum(wid * MSE_CHUNK, N - MSE_CHUNK), 8)
  mse_copies = [
      pltpu.async_copy(er_hbm.at[pl.ds(mse_off, MSE_CHUNK)], m0, sem_mse),
      pltpu.async_copy(ei_hbm.at[pl.ds(mse_off, MSE_CHUNK)], m1, sem_mse),
      pltpu.async_copy(ymr_hbm.at[pl.ds(mse_off, MSE_CHUNK)], m2, sem_mse),
      pltpu.async_copy(ymi_hbm.at[pl.ds(mse_off, MSE_CHUNK)], m3, sem_mse),
  ]

  col0 = s * (NB_TILE * BLK)

  def pass_copies(p, parity):
    r, co, vr_, vi_, sem = st[parity]
    off = pl.multiple_of(col0 + p * SWP, BLK)
    return [
        pltpu.make_async_copy(rows_hbm.at[:, pl.ds(off, SWP)], r, sem),
        pltpu.make_async_copy(cols_hbm.at[:, pl.ds(off, SWP)], co, sem),
        pltpu.make_async_copy(vr_hbm.at[:, pl.ds(off, SWP)], vr_, sem),
        pltpu.make_async_copy(vi_hbm.at[:, pl.ds(off, SWP)], vi_, sem),
    ]

  def fire_pass(p, parity):
    for cp in pass_copies(p, parity):
      cp.start()

  def wait_pass(p, parity):
    for cp in pass_copies(p, parity):
      cp.wait()

  fire_pass(0, 0)
  fire_pass(1, 1)

  @plsc.parallel_loop(0, NP // L, unroll=5)
  def _(k):
    off = pl.ds(k * L, L)
    accr0[off] = zeros
    acci0[off] = zeros
    accr1[off] = zeros
    acci1[off] = zeros

  with jax.named_scope("x_wait"):
    for cp in x_copies:
      cp.wait()

  def phase1_block(r_ref, c_ref, vr_ref, vi_ref, nvregs):
    for b_local in range(2):
      brow16 = jnp.full((L,), 2 * c + b_local, jnp.int32)
      acc_r, acc_i = accs[b_local]
      xbase = b_local * NP

      @plsc.parallel_loop(0, nvregs, unroll=4)
      def _(t):
        idx16 = t * L + iota
        rowg = plsc.load_gather(r_ref, [brow16, idx16])
        colg = plsc.load_gather(c_ref, [brow16, idx16])
        wr = plsc.load_gather(vr_ref, [brow16, idx16])
        wi = plsc.load_gather(vi_ref, [brow16, idx16])
        xcr = plsc.load_gather(xr2, [colg + xbase])
        xci = plsc.load_gather(xi2, [colg + xbase])
        ar = wr * xcr - wi * xci
        ai = wr * xci + wi * xcr
        plsc.addupdate_scatter(acc_r, [rowg], ar)
        plsc.addupdate_scatter(acc_i, [rowg], ai)

  def pass_triple(k, _):
    p0 = k * 3
    for j in range(3):
      p = p0 + j

      @pl.when(p + 2 < NPASS)
      def _():
        fire_pass(p + 2, (j + 2) % 3)

      with jax.named_scope("st_wait"):
        wait_pass(p, j)
      r_ref, c_ref, vr_ref, vi_ref, _ = st[j]
      with jax.named_scope("phase1"):
        phase1_block(r_ref, c_ref, vr_ref, vi_ref, SWP // L)
    return 0

  lax.fori_loop(0, NPASS // 3, pass_triple, 0)

  @pl.when(s == 0)
  def _():
    pltpu.sync_copy(rows_hbm.at[:, pl.ds(TAIL_OFF, TAIL_W)],
                    str0.at[:, pl.ds(0, TAIL_W)])
    pltpu.sync_copy(cols_hbm.at[:, pl.ds(TAIL_OFF, TAIL_W)],
                    stc0.at[:, pl.ds(0, TAIL_W)])
    pltpu.sync_copy(vr_hbm.at[:, pl.ds(TAIL_OFF, TAIL_W)],
                    stvr0.at[:, pl.ds(0, TAIL_W)])
    pltpu.sync_copy(vi_hbm.at[:, pl.ds(TAIL_OFF, TAIL_W)],
                    stvi0.at[:, pl.ds(0, TAIL_W)])
    phase1_block(str0, stc0, stvr0, stvi0, TAIL_W // L)

  row_base = s * SLICE
  b_off = pl.multiple_of(jnp.minimum(row_base, NP - SLICE), 8)

  def b_slices(bi):
    boff = pl.multiple_of(bi * NP, 8) + b_off
    return (br_hbm.at[pl.ds(boff, SLICE)], bi_hbm.at[pl.ds(boff, SLICE)])

  def pub_off(tile, slot):
    return pl.multiple_of(((c * NS + tile) * 4 + slot) * NP, 8)

  with jax.named_scope("publish"):
    pubs = [
        pltpu.async_copy(accr0, exch_hbm.at[pl.ds(pub_off(s, 0), NP)], sem_b),
        pltpu.async_copy(acci0, exch_hbm.at[pl.ds(pub_off(s, 1), NP)], sem_b),
        pltpu.async_copy(accr1, exch_hbm.at[pl.ds(pub_off(s, 2), NP)], sem_b),
        pltpu.async_copy(acci1, exch_hbm.at[pl.ds(pub_off(s, 3), NP)], sem_b),
    ]
    for cp in pubs:
      cp.wait()
    plsc.subcore_barrier()

  b_shift = row_base - b_off
  loss_vec = zeros
  pairs = [(q, h) for q in range(4) for h in range(2)]
  tmps = [tmpa, tmpb]
  semt = [sem_t, sem_t2]

  def pair_copies(i):
    q, h = pairs[i]
    buf = tmps[i % 2]
    return [
        pltpu.make_async_copy(
            exch_hbm.at[pl.ds(
                pl.multiple_of(pub_off(h * 8 + j, q) + b_off, 8), SLICE)],
            buf.at[pl.ds(j * SLICE, SLICE)], semt[i % 2])
        for j in range(8)
    ]

  for cp in pair_copies(0):
    cp.start()
  bcp = [pltpu.async_copy(b_slices(2 * c)[0], brv, sem_b),
         pltpu.async_copy(b_slices(2 * c)[1], biv, sem_b)]

  for i, (q, h) in enumerate(pairs):
    if i + 1 < len(pairs):
      for cp in pair_copies(i + 1):
        cp.start()
    with jax.named_scope("tmp_copy"):
      for cp in pair_copies(i):
        cp.wait()
    buf = tmps[i % 2]
    b_ref = brv if q % 2 == 0 else biv

    if h == 0:
      @plsc.parallel_loop(0, SLICE // L)
      def half0(k):
        koff = jnp.minimum(b_shift + k * L, SLICE - L)
        part = buf[pl.ds(koff, L)]
        for j in range(1, 8):
          part = part + buf[pl.ds(j * SLICE + koff, L)]
        axbuf[pl.ds(k * L, L)] = part
    else:
      if q == 1:
        pass
      if q == 0:
        with jax.named_scope("b_wait"):
          for cp in bcp:
            cp.wait()

      @plsc.parallel_loop(0, SLICE // L, carry=loss_vec)
      def half1(k, acc):
        koff = jnp.minimum(b_shift + k * L, SLICE - L)
        part = buf[pl.ds(koff, L)]
        for j in range(1, 8):
          part = part + buf[pl.ds(j * SLICE + koff, L)]
        ax = axbuf[pl.ds(k * L, L)] + part
        r = b_ref[pl.ds(koff, L)] - ax
        row_vec = row_base + k * L + iota
        return acc + jnp.where(row_vec < NP, r * r, jnp.float32(0.0))
      loss_vec = half1

      if q == 1:
        bcp = [pltpu.async_copy(b_slices(2 * c + 1)[0], brv, sem_b),
               pltpu.async_copy(b_slices(2 * c + 1)[1], biv, sem_b)]
        with jax.named_scope("b_wait"):
          for cp in bcp:
            cp.wait()

  mse_shift = wid * MSE_CHUNK - mse_off
  with jax.named_scope("mse_wait"):
    for cp in mse_copies:
      cp.wait()

  @plsc.parallel_loop(0, MSE_CHUNK // L, carry=loss_vec)
  def mse_body(k, acc):
    roff = jnp.minimum(mse_shift + k * L, MSE_CHUNK - L)
    off = pl.ds(roff, L)
    dr = m0[off] - m2[off]
    di = m1[off] - m3[off]
    contrib = dr * dr + di * di
    elem = wid * MSE_CHUNK + k * L + iota
    return acc + jnp.where(elem < N, contrib, jnp.float32(0.0))
  loss_vec = mse_body

  outv[...] = loss_vec * jnp.float32(0.5 / N)
  pltpu.sync_copy(outv, out_hbm.at[wid])


@jax.jit
def _run(er, ei, ymr, ymi, rows, cols, vr, vi, br, bi):
  mesh = plsc.VectorSubcoreMesh(
      core_axis_name="c", subcore_axis_name="s",
      num_cores=NC, num_subcores=NS)
  f = pl.kernel(
      _sc_body,
      out_type=jax.ShapeDtypeStruct((NC * NS, L), jnp.float32),
      mesh=mesh,
      compiler_params=pltpu.CompilerParams(needs_layout_passes=False),
      scratch_types=[
          pltpu.HBM((NC * NS * 4 * NP,), jnp.float32),
          pltpu.VMEM((B, SWP), jnp.int32),
          pltpu.VMEM((B, SWP), jnp.int32),
          pltpu.VMEM((B, SWP), jnp.float32),
          pltpu.VMEM((B, SWP), jnp.float32),
          pltpu.VMEM((B, SWP), jnp.int32),
          pltpu.VMEM((B, SWP), jnp.int32),
          pltpu.VMEM((B, SWP), jnp.float32),
          pltpu.VMEM((B, SWP), jnp.float32),
          pltpu.VMEM((B, SWP), jnp.int32),
          pltpu.VMEM((B, SWP), jnp.int32),
          pltpu.VMEM((B, SWP), jnp.float32),
          pltpu.VMEM((B, SWP), jnp.float32),
          pltpu.VMEM((NP,), jnp.float32),
          pltpu.VMEM((NP,), jnp.float32),
          pltpu.VMEM((NP,), jnp.float32),
          pltpu.VMEM((NP,), jnp.float32),
          pltpu.VMEM((2 * NP,), jnp.float32),
          pltpu.VMEM((2 * NP,), jnp.float32),
          pltpu.VMEM((8 * SLICE,), jnp.float32),
          pltpu.VMEM((8 * SLICE,), jnp.float32),
          pltpu.VMEM((SLICE,), jnp.float32),
          pltpu.VMEM((MSE_CHUNK,), jnp.float32),
          pltpu.VMEM((MSE_CHUNK,), jnp.float32),
          pltpu.VMEM((MSE_CHUNK,), jnp.float32),
          pltpu.VMEM((MSE_CHUNK,), jnp.float32),
          pltpu.VMEM((SLICE,), jnp.float32),
          pltpu.VMEM((SLICE,), jnp.float32),
          pltpu.VMEM((L,), jnp.float32),
          pltpu.SemaphoreType.DMA,
          pltpu.SemaphoreType.DMA,
          pltpu.SemaphoreType.DMA,
          pltpu.SemaphoreType.DMA,
          pltpu.SemaphoreType.DMA,
          pltpu.SemaphoreType.DMA,
          pltpu.SemaphoreType.DMA,
          pltpu.SemaphoreType.DMA,
      ],
  )
  return f(er, ei, ymr, ymi, rows, cols, vr, vi, br, bi)


def kernel(E_real, E_imag, batch_y, k_all, node_batch, A_rows, A_cols,
           A_vals_real, A_vals_imag, b_real, b_imag):
  del k_all, node_batch
  partials = _run(E_real, E_imag, batch_y[:, 0], batch_y[:, 1],
                  A_rows, A_cols, A_vals_real, A_vals_imag,
                  b_real.reshape(-1), b_imag.reshape(-1))
  return jnp.sum(partials)
